# Initial kernel scaffold; baseline (speedup 1.0000x reference)
import jax
import jax.numpy as jnp
from jax.experimental import pallas as pl


def kernel(x, edge_index, batch, eps_0, w1_0, b1_0, w2_0, b2_0, g_0, bt_0, eps_1, w1_1, b1_1, w2_1, b2_1, g_1, bt_1, eps_2, w1_2, b1_2, w2_2, b2_2, g_2, bt_2, hw1, hb1, hw2, hb2):
    raise NotImplementedError("write your pallas kernel here")


# speedup vs baseline: 4.2111x; 4.2111x over previous
import functools

import jax
import jax.numpy as jnp
from jax import lax
from jax.experimental import pallas as pl
from jax.experimental.pallas import tpu as pltpu
from jax.experimental.pallas import tpu_sc as plsc

N = 10000
E = 320000
D = 128
G = 64
BN_EPS = 1e-5

NC = 2
NS = 16
NW = NC * NS
EPW = E // NW
CH = -(-EPW // 128)
EPW_PAD = CH * 128
E_PAD = EPW_PAD * NW
N_PAD = 10112
ZROWS = N_PAD // NS
OROWS = N_PAD // NS


def _agg_body(x_hbm, src_hbm, dst_hbm, zeros_hbm, out_hbm,
              src_v, dst_v, rows_v, agg_sh, sem):
    c = lax.axis_index("c")
    s = lax.axis_index("s")
    w = c * NS + s
    pltpu.sync_copy(zeros_hbm, agg_sh.at[pl.ds(s * ZROWS, ZROWS)])
    pltpu.sync_copy(src_hbm.at[w], src_v)
    pltpu.sync_copy(dst_hbm.at[w], dst_v)
    plsc.subcore_barrier()

    def chunk(j, carry):
        pltpu.async_copy(x_hbm.at[src_v.at[j]], rows_v, sem).wait()
        pltpu.sync_copy(rows_v, agg_sh.at[dst_v.at[j]], add=True)
        return carry

    lax.fori_loop(0, CH, chunk, 0)
    plsc.subcore_barrier()
    pltpu.sync_copy(agg_sh.at[pl.ds(s * OROWS, OROWS)],
                    out_hbm.at[c, pl.ds(s * OROWS, OROWS)])


_agg = pl.kernel(
    _agg_body,
    out_type=jax.ShapeDtypeStruct((NC, N_PAD, D), jnp.float32),
    mesh=plsc.VectorSubcoreMesh(core_axis_name="c", subcore_axis_name="s",
                                num_cores=NC, num_subcores=NS),
    scratch_types=[
        pltpu.VMEM((CH, 128), jnp.int32),
        pltpu.VMEM((CH, 128), jnp.int32),
        pltpu.VMEM((128, D), jnp.float32),
        pltpu.VMEM_SHARED((N_PAD, D), jnp.float32),
        pltpu.SemaphoreType.DMA,
    ],
)


def _mlp_body(x_ref, a_ref, ope_ref, w1_ref, b1_ref, w2_ref, b2_ref,
              g_ref, bt_ref, o_ref):
    h = x_ref[...] * ope_ref[...] + a_ref[0, :N] + a_ref[1, :N]
    h = jnp.maximum(
        jnp.dot(h, w1_ref[...], preferred_element_type=jnp.float32)
        + b1_ref[...], 0.0)
    h = jnp.dot(h, w2_ref[...], preferred_element_type=jnp.float32) + b2_ref[...]
    mu = jnp.mean(h, axis=0, keepdims=True)
    var = jnp.mean((h - mu) * (h - mu), axis=0, keepdims=True)
    h = (h - mu) * lax.rsqrt(var + BN_EPS) * g_ref[...] + bt_ref[...]
    o_ref[...] = jnp.maximum(h, 0.0)


_mlp = pl.pallas_call(
    _mlp_body,
    out_shape=jax.ShapeDtypeStruct((N, D), jnp.float32),
)


def _final_body(x_ref, a_ref, ope_ref, w1_ref, b1_ref, w2_ref, b2_ref,
                g_ref, bt_ref, batch_ref, hw1_ref, hb1_ref, hw2_ref, hb2_ref,
                o_ref):
    h = x_ref[...] * ope_ref[...] + a_ref[0, :N] + a_ref[1, :N]
    h = jnp.maximum(
        jnp.dot(h, w1_ref[...], preferred_element_type=jnp.float32)
        + b1_ref[...], 0.0)
    h = jnp.dot(h, w2_ref[...], preferred_element_type=jnp.float32) + b2_ref[...]
    mu = jnp.mean(h, axis=0, keepdims=True)
    var = jnp.mean((h - mu) * (h - mu), axis=0, keepdims=True)
    h = (h - mu) * lax.rsqrt(var + BN_EPS) * g_ref[...] + bt_ref[...]
    h = jnp.maximum(h, 0.0)
    gi = lax.broadcasted_iota(jnp.int32, (N, G), 1)
    oh = (batch_ref[...] == gi).astype(jnp.float32)
    s = lax.dot_general(oh, h, (((0,), (0,)), ((), ())),
                        preferred_element_type=jnp.float32)
    cnt = lax.dot_general(oh, jnp.ones((N, 1), jnp.float32),
                          (((0,), (0,)), ((), ())),
                          preferred_element_type=jnp.float32)
    pooled = s / jnp.maximum(cnt, 1.0)
    hh = jnp.maximum(
        jnp.dot(pooled, hw1_ref[...], preferred_element_type=jnp.float32)
        + hb1_ref[...], 0.0)
    o_ref[...] = (jnp.dot(hh, hw2_ref[...], preferred_element_type=jnp.float32)
                  + hb2_ref[...])


_final = pl.pallas_call(
    _final_body,
    out_shape=jax.ShapeDtypeStruct((G, D), jnp.float32),
)


def kernel(x, edge_index, batch,
           eps_0, w1_0, b1_0, w2_0, b2_0, g_0, bt_0,
           eps_1, w1_1, b1_1, w2_1, b2_1, g_1, bt_1,
           eps_2, w1_2, b1_2, w2_2, b2_2, g_2, bt_2,
           hw1, hb1, hw2, hb2):
    pad = E_PAD - E
    src_p = jnp.concatenate(
        [edge_index[0], jnp.zeros((pad,), jnp.int32)]).reshape(NW, CH, 128)
    dst_p = jnp.concatenate(
        [edge_index[1], jnp.full((pad,), N, jnp.int32)]).reshape(NW, CH, 128)
    zeros = jnp.zeros((ZROWS, D), jnp.float32)
    batch2d = batch.reshape(N, 1)

    layers = [
        (eps_0, w1_0, b1_0, w2_0, b2_0, g_0, bt_0),
        (eps_1, w1_1, b1_1, w2_1, b2_1, g_1, bt_1),
        (eps_2, w1_2, b1_2, w2_2, b2_2, g_2, bt_2),
    ]

    for l, (eps, w1, b1, w2, b2, g, bt) in enumerate(layers):
        a = _agg(x, src_p, dst_p, zeros)
        ope = (1.0 + eps).reshape(1, 1).astype(jnp.float32)
        args = (x, a, ope, w1, b1.reshape(1, D), w2, b2.reshape(1, D),
                g.reshape(1, D), bt.reshape(1, D))
        if l < 2:
            x = _mlp(*args)
        else:
            return _final(*args, batch2d, hw1, hb1.reshape(1, D),
                          hw2, hb2.reshape(1, D))
